# Initial kernel scaffold; baseline (speedup 1.0000x reference)
import jax
import jax.numpy as jnp
from jax.experimental import pallas as pl


def kernel(features, edge_index, edge_types, W1, al1, ar1, b1, W2, al2, ar2, b2):
    raise NotImplementedError("write your pallas kernel here")


# speedup vs baseline: 29.1823x; 29.1823x over previous
import functools

import jax
import jax.numpy as jnp
from jax import lax
from jax.experimental import pallas as pl
from jax.experimental.pallas import tpu as pltpu
from jax.experimental.pallas import tpu_sc as plsc

NN = 10000
EE = 320000
NH = 8
HD = 16
FW = NH * HD
RB = 1000
NGB = NN // RB
NC = 2
NS = 16
WB = 32
WB2 = 128
EPC = EE // NC
NWIN = EPC // WB
NWIN2 = EPC // WB2
NNP = 10240
RPT = NNP // NS
NSL = NNP // 8
SPT = NSL // NS

_HIGHEST = lax.Precision.HIGHEST


def _tc1_body(x_ref, w1_ref, alrow_ref, arrow_ref, z_ref, meta_ref, gmax_ref):
    pid = pl.program_id(0)
    zb = jnp.dot(x_ref[...], w1_ref[...], preferred_element_type=jnp.float32,
                 precision=_HIGHEST)
    z_ref[...] = zb
    d_i = lax.broadcasted_iota(jnp.int32, (FW, 16), 0)
    j_i = lax.broadcasted_iota(jnp.int32, (FW, 16), 1)
    s0 = ((d_i // HD) == j_i).astype(jnp.float32)
    s1 = ((d_i // HD) == (j_i - NH)).astype(jnp.float32)
    meta = (jnp.dot(zb * alrow_ref[...], s0, preferred_element_type=jnp.float32,
                    precision=_HIGHEST)
            + jnp.dot(zb * arrow_ref[...], s1, preferred_element_type=jnp.float32,
                      precision=_HIGHEST))
    meta_ref[...] = meta
    bm = jnp.max(meta, axis=0, keepdims=True)

    @pl.when(pid == 0)
    def _():
        gmax_ref[...] = bm

    @pl.when(pid != 0)
    def _():
        gmax_ref[...] = jnp.maximum(gmax_ref[...], bm)


def _tc_dense1(x, w1, alrow, arrow):
    return pl.pallas_call(
        _tc1_body,
        grid=(NGB,),
        in_specs=[
            pl.BlockSpec((RB, FW), lambda i: (i, 0)),
            pl.BlockSpec((FW, FW), lambda i: (0, 0)),
            pl.BlockSpec((1, FW), lambda i: (0, 0)),
            pl.BlockSpec((1, FW), lambda i: (0, 0)),
        ],
        out_specs=[
            pl.BlockSpec((RB, FW), lambda i: (i, 0)),
            pl.BlockSpec((RB, 16), lambda i: (i, 0)),
            pl.BlockSpec((1, 16), lambda i: (0, 0)),
        ],
        out_shape=[
            jax.ShapeDtypeStruct((NN, FW), jnp.float32),
            jax.ShapeDtypeStruct((NNP, 16), jnp.float32),
            jax.ShapeDtypeStruct((1, 16), jnp.float32),
        ],
    )(x, w1, alrow, arrow)


def _sc_layer1_body(z_hbm, mslot_hbm, gm8_hbm, src_hbm, dst_hbm,
                    outp_hbm, denp_hbm,
                    outacc, denslot, idxs_v, idxd_v, idxs8_v, idxd8_v,
                    zwin, mswin, mdwin, exslot, contrib, gmb,
                    sem0, sem1, sem2):
    c = lax.axis_index("c")
    s = lax.axis_index("s")
    zv16 = jnp.zeros((16,), jnp.float32)
    lane = lax.iota(jnp.int32, 16)
    perm = (lane + 8) % 16
    msk8 = lane < NH

    def _zw(i, _):
        for k in range(FW // 16):
            exslot[i, pl.ds(k * 16, 16)] = zv16
        return 0
    lax.fori_loop(0, WB, _zw, 0)

    rowbase = s * RPT
    for k in range(RPT // WB):
        pltpu.sync_copy(exslot, outacc.at[pl.ds(rowbase + k * WB, WB)])
    slotbase = s * SPT
    for k in range(SPT // 16):
        pltpu.sync_copy(exslot.at[pl.ds(0, 16)],
                        denslot.at[pl.ds(slotbase + k * 16, 16)])

    pltpu.sync_copy(gm8_hbm, gmb)
    gvec = gmb[0, pl.ds(0, 16)]

    plsc.subcore_barrier()

    ebase = c * EPC

    def window_body(j, _):
        base = ebase + (s + j * NS) * WB
        pltpu.sync_copy(src_hbm.at[pl.ds(base, WB)], idxs_v)
        pltpu.sync_copy(dst_hbm.at[pl.ds(base, WB)], idxd_v)
        for g in range(WB // 16):
            sv = idxs_v[pl.ds(g * 16, 16)]
            dv = idxd_v[pl.ds(g * 16, 16)]
            idxs8_v[pl.ds(g * 16, 16)] = sv >> 3
            idxd8_v[pl.ds(g * 16, 16)] = dv >> 3
        cp1 = pltpu.async_copy(z_hbm.at[idxs_v], zwin, sem0)
        cp2 = pltpu.async_copy(mslot_hbm.at[idxs8_v], mswin, sem1)
        cp3 = pltpu.async_copy(mslot_hbm.at[idxd8_v], mdwin, sem2)
        cp1.wait()
        cp2.wait()
        cp3.wait()

        for g in range(WB // 16):
            soffv = (idxs_v[pl.ds(g * 16, 16)] & 7) * 16
            doffv = (idxd_v[pl.ds(g * 16, 16)] & 7) * 16

            def edge_body(i, _):
                row = jnp.full((16,), g * 16, jnp.int32) + i
                isel = jnp.full((16,), i, jnp.int32)
                so = jnp.take_along_axis(soffv, isel, axis=0,
                                         mode="promise_in_bounds")
                do = jnp.take_along_axis(doffv, isel, axis=0,
                                         mode="promise_in_bounds")
                ms = plsc.load_gather(mswin, [row, so + lane])
                md = plsc.load_gather(mdwin, [row, do + lane])
                mdp = jnp.take_along_axis(md, perm, axis=0,
                                          mode="promise_in_bounds")
                t = ms + mdp
                u = jnp.where(t >= 0.0, t, 0.2 * t)
                craw = gvec + mdp
                cl = jnp.where(craw >= 0.0, craw, 0.2 * craw)
                ex = jnp.where(msk8, jnp.exp(u - cl), 0.0)
                plsc.store_scatter(exslot, [row, do + lane], ex)
                ri = g * 16 + i
                for h in range(NH):
                    hsel = jnp.full((16,), h, jnp.int32)
                    exs = jnp.take_along_axis(ex, hsel, axis=0,
                                              mode="promise_in_bounds")
                    contrib[ri, pl.ds(h * HD, HD)] = (
                        exs * zwin[ri, pl.ds(h * HD, HD)])
                return 0

            lax.fori_loop(0, 16, edge_body, 0)

        pltpu.sync_copy(contrib, outacc.at[idxd_v], add=True)
        pltpu.sync_copy(exslot, denslot.at[idxd8_v], add=True)

        for g in range(WB // 16):
            doffv = (idxd_v[pl.ds(g * 16, 16)] & 7) * 16

            def zero_body(i, _):
                row = jnp.full((16,), g * 16, jnp.int32) + i
                isel = jnp.full((16,), i, jnp.int32)
                do = jnp.take_along_axis(doffv, isel, axis=0,
                                         mode="promise_in_bounds")
                plsc.store_scatter(exslot, [row, do + lane], zv16)
                return 0

            lax.fori_loop(0, 16, zero_body, 0)
        return 0

    ntile = (NWIN - s + NS - 1) // NS
    lax.fori_loop(0, ntile, window_body, 0)

    plsc.subcore_barrier()

    def _dump(k, _):
        r = rowbase + k * WB
        pltpu.sync_copy(outacc.at[pl.ds(r, WB)], zwin)
        pltpu.sync_copy(zwin, outp_hbm.at[pl.ds(c * NNP + r, WB)])
        return 0
    lax.fori_loop(0, RPT // WB, _dump, 0)

    def _dump2(k, _):
        r = slotbase + k * 16
        pltpu.sync_copy(denslot.at[pl.ds(r, 16)], zwin.at[pl.ds(0, 16)])
        pltpu.sync_copy(zwin.at[pl.ds(0, 16)],
                        denp_hbm.at[pl.ds(c * NSL + r, 16)])
        return 0
    lax.fori_loop(0, SPT // 16, _dump2, 0)


def _tc2_body(o0_ref, o1_ref, d0_ref, d1_ref, b1_ref, w2row_ref, al2_ref,
              ar2_ref, meta2_ref, g2_ref):
    pid = pl.program_id(0)
    den16 = d0_ref[0] + d1_ref[0]
    j_i = lax.broadcasted_iota(jnp.int32, (16, FW), 0)
    l_i = lax.broadcasted_iota(jnp.int32, (16, FW), 1)
    tmat = ((l_i // HD) == j_i).astype(jnp.float32)
    den128 = jnp.dot(den16, tmat, preferred_element_type=jnp.float32,
                     precision=_HIGHEST)
    outun = o0_ref[0] + o1_ref[0]
    h = jnp.maximum(outun / (den128 + 1e-9) + b1_ref[...], 0.0)
    z2 = jnp.sum(h * w2row_ref[...], axis=1, keepdims=True)
    al2v = al2_ref[0, 0]
    ar2v = ar2_ref[0, 0]
    li = lax.broadcasted_iota(jnp.int32, (1, 16), 1)
    sel = jnp.where(li == 0, 1.0,
                    jnp.where(li == 1, al2v, jnp.where(li == 2, ar2v, 0.0)))
    meta2 = z2 * sel
    meta2_ref[...] = meta2
    bm = jnp.max(meta2, axis=0, keepdims=True)

    @pl.when(pid == 0)
    def _():
        g2_ref[...] = bm

    @pl.when(pid != 0)
    def _():
        g2_ref[...] = jnp.maximum(g2_ref[...], bm)


def _tc_dense2(outp3, denp3, b1row, w2row, al2m, ar2m):
    return pl.pallas_call(
        _tc2_body,
        grid=(NGB,),
        in_specs=[
            pl.BlockSpec((1, RB, FW), lambda i: (0, i, 0)),
            pl.BlockSpec((1, RB, FW), lambda i: (1, i, 0)),
            pl.BlockSpec((1, RB, 16), lambda i: (0, i, 0)),
            pl.BlockSpec((1, RB, 16), lambda i: (1, i, 0)),
            pl.BlockSpec((1, FW), lambda i: (0, 0)),
            pl.BlockSpec((1, FW), lambda i: (0, 0)),
            pl.BlockSpec((1, 1), lambda i: (0, 0)),
            pl.BlockSpec((1, 1), lambda i: (0, 0)),
        ],
        out_specs=[
            pl.BlockSpec((RB, 16), lambda i: (i, 0)),
            pl.BlockSpec((1, 16), lambda i: (0, 0)),
        ],
        out_shape=[
            jax.ShapeDtypeStruct((NNP, 16), jnp.float32),
            jax.ShapeDtypeStruct((1, 16), jnp.float32),
        ],
    )(outp3, outp3, denp3, denp3, b1row, w2row, al2m, ar2m)


def _sc_layer2_body(m2slot_hbm, g2m8_hbm, src_hbm, dst_hbm, acc2p_hbm,
                    acc2, idxs_v, idxd_v, idxs8_v, idxd8_v, m2s, m2d, ctr,
                    g2b, sem0, sem1):
    c = lax.axis_index("c")
    s = lax.axis_index("s")
    zv16 = jnp.zeros((16,), jnp.float32)
    lane = lax.iota(jnp.int32, 16)

    def _zc(i, _):
        for k in range(FW // 16):
            ctr[i, pl.ds(k * 16, 16)] = zv16
        return 0
    lax.fori_loop(0, WB2, _zc, 0)

    slotbase = s * SPT
    for k in range(SPT // 16):
        pltpu.sync_copy(ctr.at[pl.ds(0, 16)],
                        acc2.at[pl.ds(slotbase + k * 16, 16)])

    pltpu.sync_copy(g2m8_hbm, g2b)
    g2row = g2b[0, pl.ds(0, 16)]
    one = jnp.full((16,), 1, jnp.int32)
    g2s = jnp.take_along_axis(g2row, one, axis=0, mode="promise_in_bounds")

    plsc.subcore_barrier()

    ebase = c * EPC

    def window_body(j, _):
        base = ebase + (s + j * NS) * WB2
        pltpu.sync_copy(src_hbm.at[pl.ds(base, WB2)], idxs_v)
        pltpu.sync_copy(dst_hbm.at[pl.ds(base, WB2)], idxd_v)
        for g in range(WB2 // 16):
            sv = idxs_v[pl.ds(g * 16, 16)]
            dv = idxd_v[pl.ds(g * 16, 16)]
            idxs8_v[pl.ds(g * 16, 16)] = sv >> 3
            idxd8_v[pl.ds(g * 16, 16)] = dv >> 3
        cp1 = pltpu.async_copy(m2slot_hbm.at[idxs8_v], m2s, sem0)
        cp2 = pltpu.async_copy(m2slot_hbm.at[idxd8_v], m2d, sem1)
        cp1.wait()
        cp2.wait()

        for g in range(WB2 // 16):
            rows = lane + g * 16
            soff = (idxs_v[pl.ds(g * 16, 16)] & 7) * 16
            doff = (idxd_v[pl.ds(g * 16, 16)] & 7) * 16
            z2v = plsc.load_gather(m2s, [rows, soff])
            el2v = plsc.load_gather(m2s, [rows, soff + 1])
            er2v = plsc.load_gather(m2d, [rows, doff + 2])
            t = el2v + er2v
            u = jnp.where(t >= 0.0, t, 0.2 * t)
            craw = g2s + er2v
            cl = jnp.where(craw >= 0.0, craw, 0.2 * craw)
            ex = jnp.exp(u - cl)
            plsc.store_scatter(ctr, [rows, doff], ex * z2v)
            plsc.store_scatter(ctr, [rows, doff + 1], ex)

        pltpu.sync_copy(ctr, acc2.at[idxd8_v], add=True)

        for g in range(WB2 // 16):
            rows = lane + g * 16
            doff = (idxd_v[pl.ds(g * 16, 16)] & 7) * 16
            plsc.store_scatter(ctr, [rows, doff], zv16)
            plsc.store_scatter(ctr, [rows, doff + 1], zv16)
        return 0

    ntile = (NWIN2 - s + NS - 1) // NS
    lax.fori_loop(0, ntile, window_body, 0)

    plsc.subcore_barrier()

    def _dump(k, _):
        r = slotbase + k * 16
        pltpu.sync_copy(acc2.at[pl.ds(r, 16)], ctr.at[pl.ds(0, 16)])
        pltpu.sync_copy(ctr.at[pl.ds(0, 16)],
                        acc2p_hbm.at[pl.ds(c * NSL + r, 16)])
        return 0
    lax.fori_loop(0, SPT // 16, _dump, 0)


@functools.lru_cache(maxsize=1)
def _sc_kernels():
    mesh = plsc.VectorSubcoreMesh(core_axis_name="c", subcore_axis_name="s",
                                  num_cores=NC, num_subcores=NS)
    params = pltpu.CompilerParams(needs_layout_passes=False)
    sc1 = pl.kernel(
        _sc_layer1_body,
        out_type=[
            jax.ShapeDtypeStruct((NC * NNP, FW), jnp.float32),
            jax.ShapeDtypeStruct((NC * NSL, FW), jnp.float32),
        ],
        mesh=mesh,
        compiler_params=params,
        scratch_types=[
            pltpu.VMEM_SHARED((NNP, FW), jnp.float32),
            pltpu.VMEM_SHARED((NSL, FW), jnp.float32),
            pltpu.VMEM((WB,), jnp.int32),
            pltpu.VMEM((WB,), jnp.int32),
            pltpu.VMEM((WB,), jnp.int32),
            pltpu.VMEM((WB,), jnp.int32),
            pltpu.VMEM((WB, FW), jnp.float32),
            pltpu.VMEM((WB, FW), jnp.float32),
            pltpu.VMEM((WB, FW), jnp.float32),
            pltpu.VMEM((WB, FW), jnp.float32),
            pltpu.VMEM((WB, FW), jnp.float32),
            pltpu.VMEM((8, FW), jnp.float32),
            pltpu.SemaphoreType.DMA,
            pltpu.SemaphoreType.DMA,
            pltpu.SemaphoreType.DMA,
        ],
    )
    sc2 = pl.kernel(
        _sc_layer2_body,
        out_type=[jax.ShapeDtypeStruct((NC * NSL, FW), jnp.float32)],
        mesh=mesh,
        compiler_params=params,
        scratch_types=[
            pltpu.VMEM_SHARED((NSL, FW), jnp.float32),
            pltpu.VMEM((WB2,), jnp.int32),
            pltpu.VMEM((WB2,), jnp.int32),
            pltpu.VMEM((WB2,), jnp.int32),
            pltpu.VMEM((WB2,), jnp.int32),
            pltpu.VMEM((WB2, FW), jnp.float32),
            pltpu.VMEM((WB2, FW), jnp.float32),
            pltpu.VMEM((WB2, FW), jnp.float32),
            pltpu.VMEM((8, FW), jnp.float32),
            pltpu.SemaphoreType.DMA,
            pltpu.SemaphoreType.DMA,
        ],
    )
    return sc1, sc2


def _tc3_body(a0_ref, a1_ref, b2_ref, out_ref):
    li = lax.broadcasted_iota(jnp.int32, (1, 16), 1)
    m0 = (li == 0).astype(jnp.float32)
    m1 = (li == 1).astype(jnp.float32)
    a = a0_ref[0] + a1_ref[0]
    num = jnp.sum(a * m0, axis=1, keepdims=True)
    den = jnp.sum(a * m1, axis=1, keepdims=True)
    r = num / (den + 1e-9) + b2_ref[0, 0]
    out_ref[...] = jax.nn.sigmoid(r)


def _tc_final(acc2p3, b2m):
    return pl.pallas_call(
        _tc3_body,
        grid=(NGB,),
        in_specs=[
            pl.BlockSpec((1, RB, 16), lambda i: (0, i, 0)),
            pl.BlockSpec((1, RB, 16), lambda i: (1, i, 0)),
            pl.BlockSpec((1, 1), lambda i: (0, 0)),
        ],
        out_specs=pl.BlockSpec((RB, 1), lambda i: (i, 0)),
        out_shape=jax.ShapeDtypeStruct((NN, 1), jnp.float32),
    )(acc2p3, acc2p3, b2m)


def kernel(features, edge_index, edge_types, W1, al1, ar1, b1, W2, al2, ar2, b2):
    del edge_types
    src = edge_index[0]
    dst = edge_index[1]
    alrow = al1.reshape(1, FW)
    arrow = ar1.reshape(1, FW)
    sc1, sc2 = _sc_kernels()

    z, meta, gmax16 = _tc_dense1(features, W1, alrow, arrow)
    mslot = meta.reshape(NSL, FW)
    gm8 = jnp.zeros((8, FW), jnp.float32).at[0, :16].set(gmax16[0])
    outp, denp = sc1(z, mslot, gm8, src, dst)

    meta2, g2 = _tc_dense2(outp.reshape(NC, NNP, FW),
                           denp.reshape(NC, NNP, 16),
                           b1.reshape(1, FW), W2.reshape(1, FW),
                           al2.reshape(1, 1), ar2.reshape(1, 1))
    m2slot = meta2.reshape(NSL, FW)
    g2m8 = jnp.zeros((8, FW), jnp.float32).at[0, :16].set(g2[0])
    (acc2p,) = sc2(m2slot, g2m8, src, dst)

    return _tc_final(acc2p.reshape(NC, NNP, 16), b2.reshape(1, 1))
